# baseline (speedup 1.0000x reference)
import jax
import jax.numpy as jnp
from jax import lax
from jax.experimental import pallas as pl
from jax.experimental.pallas import tpu as pltpu
from jax.experimental.pallas import tpu_sc as plsc

NC, NS = 2, 16
NW = NC * NS
BATCH, SEQ, EMBED = 4096, 200, 64
BPW = BATCH // NW
CH = 104
NCH = 2
RU = 8
L = 16


def _sc_body(end_hbm, table_hbm, wb_hbm, out_hbm,
             idx_s, idx_v, rows00, rows01, rows10, rows11, wb_v, out_v,
             sem00, sem01, sem10, sem11):
    wid = lax.axis_index("s") * NC + lax.axis_index("c")
    base = wid * BPW
    pltpu.sync_copy(end_hbm.at[0, :, pl.ds(base, BPW)], idx_s)
    pltpu.sync_copy(wb_hbm, wb_v)
    w0 = wb_v[pl.ds(0, L)]
    w1 = wb_v[pl.ds(L, L)]
    w2 = wb_v[pl.ds(2 * L, L)]
    w3 = wb_v[pl.ds(3 * L, L)]
    bias = wb_v[pl.ds(4 * L, L)][0]
    lane = lax.iota(jnp.int32, L)
    lane0 = lane == 0
    ninf = jnp.full((L,), -jnp.inf, jnp.float32)

    for j in range(NCH):
        jv = jnp.full((L,), j, jnp.int32)

        def tr_body(p, carry, j=j, jv=jv):
            s = jnp.minimum(j * CH + p, SEQ - 1)
            pv = jnp.full((L,), p, jnp.int32)
            for g in range(BPW // L):
                x = idx_s[s, pl.ds(g * L, L)]
                plsc.store_scatter(idx_v, [lane + (g * L), jv, pv], x)
            return carry

        lax.fori_loop(0, CH, tr_body, 0)

    bufs = ((rows00, rows01, sem00, sem01),
            (rows10, rows11, sem10, sem11))

    def issue(b, slot):
        r0, r1, s0, s1 = bufs[slot]
        pltpu.async_copy(table_hbm.at[idx_v.at[b, 0]], r0, s0)
        pltpu.async_copy(table_hbm.at[idx_v.at[b, 1]], r1, s1)

    def reduce_chunk(rref, acc):
        def body(i, acc):
            a0, a1, a2, a3 = acc
            for j in range(RU):
                r = i * RU + j
                a0 = jnp.maximum(a0, rref[r, pl.ds(0, L)])
                a1 = jnp.maximum(a1, rref[r, pl.ds(L, L)])
                a2 = jnp.maximum(a2, rref[r, pl.ds(2 * L, L)])
                a3 = jnp.maximum(a3, rref[r, pl.ds(3 * L, L)])
            return (a0, a1, a2, a3)
        return lax.fori_loop(0, CH // RU, body, acc)

    def consume(b, slot):
        r0, r1, s0, s1 = bufs[slot]
        pltpu.make_async_copy(table_hbm.at[idx_v.at[b, 0]], r0, s0).wait()
        pltpu.make_async_copy(table_hbm.at[idx_v.at[b, 1]], r1, s1).wait()
        acc = reduce_chunk(r0, (ninf, ninf, ninf, ninf))
        a0, a1, a2, a3 = reduce_chunk(r1, acc)
        t = a0 * w0 + a1 * w1 + a2 * w2 + a3 * w3
        s = jnp.sum(t) + bias
        plsc.store_scatter(out_v, [jnp.full((L,), b, jnp.int32)],
                           jnp.broadcast_to(s, (L,)), mask=lane0)

    issue(0, 0)
    issue(1, 1)

    def pipe(g, carry):
        b = 2 * g
        consume(b, 0)
        issue(b + 2, 0)
        consume(b + 1, 1)
        issue(b + 3, 1)
        return carry

    lax.fori_loop(0, BPW // 2 - 1, pipe, 0)
    consume(BPW - 2, 0)
    consume(BPW - 1, 1)
    pltpu.sync_copy(out_v, out_hbm.at[pl.ds(base, BPW)])


@jax.jit
def _sc_call(endings, table, wb):
    mesh = plsc.VectorSubcoreMesh(core_axis_name="c", subcore_axis_name="s")
    return pl.kernel(
        _sc_body,
        out_type=jax.ShapeDtypeStruct((BATCH,), jnp.float32),
        mesh=mesh,
        scratch_types=[
            pltpu.VMEM((SEQ, BPW), jnp.int32),
            pltpu.VMEM((BPW, NCH, CH), jnp.int32),
            pltpu.VMEM((CH, EMBED), jnp.float32),
            pltpu.VMEM((CH, EMBED), jnp.float32),
            pltpu.VMEM((CH, EMBED), jnp.float32),
            pltpu.VMEM((CH, EMBED), jnp.float32),
            pltpu.VMEM((5 * L,), jnp.float32),
            pltpu.VMEM((BPW,), jnp.float32),
            pltpu.SemaphoreType.DMA,
            pltpu.SemaphoreType.DMA,
            pltpu.SemaphoreType.DMA,
            pltpu.SemaphoreType.DMA,
        ],
        compiler_params=pltpu.CompilerParams(
            use_tc_tiling_on_sc=False, needs_layout_passes=False),
    )(endings, table, wb)


def kernel(context, endings, embed_table, fc_w, fc_b):
    wb = jnp.concatenate(
        [fc_w.reshape(EMBED), jnp.broadcast_to(fc_b, (L,))])
    return _sc_call(endings, embed_table, wb)
